# Initial kernel scaffold; baseline (speedup 1.0000x reference)
import jax
import jax.numpy as jnp
from jax.experimental import pallas as pl


def kernel(xyz, feats, W1, b1, g1, be1, W2, b2, g2, be2):
    raise NotImplementedError("write your pallas kernel here")


# speedup vs baseline: 4.7263x; 4.7263x over previous
import functools

import jax
import jax.numpy as jnp
from jax import lax
from jax.experimental import pallas as pl
from jax.experimental.pallas import tpu as pltpu
from jax.experimental.pallas import tpu_sc as plsc

_N = 8192
_M = 2048
_K = 32
_H = 128
_O = 256
_R2 = 0.1 * 0.1
_MB_Q = 8
_MB_M = 64
_NB = 512


def _fps_body(xyz_ref, xt_ref, cent_ref, cidx_ref, dist_ref):
    dist_ref[...] = jnp.full((64, 128), jnp.inf, jnp.float32)
    rows = lax.broadcasted_iota(jnp.int32, (64, 128), 0)
    cols = lax.broadcasted_iota(jnp.int32, (64, 128), 1)
    idx2d = rows * 128 + cols
    x0 = xt_ref[0, 0]
    x1 = xt_ref[0, 1]
    x2 = xt_ref[0, 2]

    def body(i, far):
        cent = xyz_ref[0, pl.ds(far, 1), :]
        cent_ref[0, pl.ds(i, 1), :] = cent
        cidx_ref[0, pl.ds(i, 1), :] = jnp.full((1, 1), far, jnp.int32)
        c0 = cent[0, 0]
        c1 = cent[0, 1]
        c2 = cent[0, 2]
        dx = x0 - c0
        dy = x1 - c1
        dz = x2 - c2
        d = (dx * dx + dy * dy) + dz * dz
        nd = jnp.minimum(dist_ref[...], d)
        dist_ref[...] = nd
        m = jnp.max(nd)
        cand = jnp.where(nd == m, idx2d, _N)
        return jnp.min(cand)

    lax.fori_loop(0, _M, body, jnp.int32(0))


def _fps(xyz, xt):
    B = xyz.shape[0]
    return pl.pallas_call(
        _fps_body,
        grid=(B,),
        in_specs=[
            pl.BlockSpec((1, _N, 3), lambda b: (b, 0, 0)),
            pl.BlockSpec((1, 3, 64, 128), lambda b: (b, 0, 0, 0)),
        ],
        out_specs=[
            pl.BlockSpec((1, _M, 3), lambda b: (b, 0, 0)),
            pl.BlockSpec((1, _M, 1), lambda b: (b, 0, 0)),
        ],
        out_shape=[
            jax.ShapeDtypeStruct((B, _M, 3), jnp.float32),
            jax.ShapeDtypeStruct((B, _M, 1), jnp.int32),
        ],
        scratch_shapes=[pltpu.VMEM((64, 128), jnp.float32)],
    )(xyz, xt)


def _ballq_body(cent_ref, cidx_ref, xf_ref, nidx_ref):
    b = pl.program_id(0)
    c = cent_ref[0]
    c0 = c[:, 0:1]
    c1 = c[:, 1:2]
    c2 = c[:, 2:3]
    x0 = xf_ref[0, 0:1, :]
    x1 = xf_ref[0, 1:2, :]
    x2 = xf_ref[0, 2:3, :]
    cs = (c0 * c0 + c1 * c1) + c2 * c2
    xs = (x0 * x0 + x1 * x1) + x2 * x2
    dot = jnp.dot(c, xf_ref[0], preferred_element_type=jnp.float32)
    d2 = cs - 2.0 * dot + xs
    valid = d2 < _R2
    iota_n = lax.broadcasted_iota(jnp.int32, (_MB_Q, _N), 1)
    keys = jnp.where(valid, iota_n, _N)
    cols = []
    for _ in range(_K):
        cur = jnp.min(keys, axis=1, keepdims=True)
        cols.append(cur)
        keys = jnp.where(keys == cur, _N, keys)
    nidx = jnp.concatenate(cols, axis=1)
    nidx = jnp.where(nidx == _N, cidx_ref[0], nidx)
    nidx_ref[0] = nidx + b * _N


def _ballq(cent, cidx, xf):
    B = cent.shape[0]
    return pl.pallas_call(
        _ballq_body,
        grid=(B, _M // _MB_Q),
        in_specs=[
            pl.BlockSpec((1, _MB_Q, 3), lambda b, m: (b, m, 0)),
            pl.BlockSpec((1, _MB_Q, 1), lambda b, m: (b, m, 0)),
            pl.BlockSpec((1, 3, _N), lambda b, m: (b, 0, 0)),
        ],
        out_specs=pl.BlockSpec((1, _MB_Q, _K), lambda b, m: (b, m, 0)),
        out_shape=jax.ShapeDtypeStruct((B, _M, _K), jnp.int32),
    )(cent, cidx, xf)


def _pmat_body(f_ref, x_ref, w1f_ref, w1x_ref, b1_ref, p_ref):
    xb = x_ref[0]
    px = (xb[:, 0:1] * w1x_ref[0:1, :]
          + xb[:, 1:2] * w1x_ref[1:2, :]
          + xb[:, 2:3] * w1x_ref[2:3, :])
    p_ref[0] = (jnp.dot(f_ref[0], w1f_ref[...],
                        preferred_element_type=jnp.float32)
                + px + b1_ref[...])


def _pmat(feats, xyz, w1f, w1x, b1_2d):
    B = feats.shape[0]
    return pl.pallas_call(
        _pmat_body,
        grid=(B, _N // _NB),
        in_specs=[
            pl.BlockSpec((1, _NB, _H), lambda b, n: (b, n, 0)),
            pl.BlockSpec((1, _NB, 3), lambda b, n: (b, n, 0)),
            pl.BlockSpec((_H, _H), lambda b, n: (0, 0)),
            pl.BlockSpec((3, _H), lambda b, n: (0, 0)),
            pl.BlockSpec((1, _H), lambda b, n: (0, 0)),
        ],
        out_specs=pl.BlockSpec((1, _NB, _H), lambda b, n: (b, n, 0)),
        out_shape=jax.ShapeDtypeStruct((B, _N, _H), jnp.float32),
    )(feats, xyz, w1f, w1x, b1_2d)


def _sc_gather(table, idx):
    R = idx.shape[0]
    NW = 32
    per_w = R // NW
    ch = 128
    iters = per_w // ch
    mesh = plsc.VectorSubcoreMesh(core_axis_name="c", subcore_axis_name="s")

    @functools.partial(
        pl.kernel,
        mesh=mesh,
        out_type=jax.ShapeDtypeStruct((R, _H), jnp.float32),
        scratch_types=[
            pltpu.VMEM((ch,), jnp.int32),
            pltpu.VMEM((ch, _H), jnp.float32),
            pltpu.SemaphoreType.DMA,
        ],
    )
    def k(table_hbm, idx_hbm, out_hbm, idx_v, rows_v, sem):
        wid = lax.axis_index("s") * 2 + lax.axis_index("c")
        base = wid * per_w

        def body(i, carry):
            o = base + i * ch
            pltpu.sync_copy(idx_hbm.at[pl.ds(o, ch)], idx_v)
            pltpu.async_copy(table_hbm.at[idx_v], rows_v, sem).wait()
            pltpu.sync_copy(rows_v, out_hbm.at[pl.ds(o, ch)])
            return carry

        lax.fori_loop(0, iters, body, jnp.int32(0))

    return k(table, idx)


def _mlp_body(g_ref, c_ref, w1x_ref, g1_ref, be1_ref, w2_ref, b2_ref,
              g2_ref, be2_ref, out_ref):
    c = c_ref[0]
    q = (c[:, 0:1] * w1x_ref[0:1, :]
         + c[:, 1:2] * w1x_ref[1:2, :]
         + c[:, 2:3] * w1x_ref[2:3, :])
    x = g_ref[0] - q[:, None, :]
    mu = jnp.mean(x, axis=-1, keepdims=True)
    var = jnp.mean((x - mu) ** 2, axis=-1, keepdims=True)
    xn = (x - mu) / jnp.sqrt(var + 1e-5) * g1_ref[...] + be1_ref[...]
    h = jnp.maximum(xn, 0.0)
    h2 = (jnp.dot(h.reshape(_MB_M * _K, _H), w2_ref[...],
                  preferred_element_type=jnp.float32) + b2_ref[...])
    mu2 = jnp.mean(h2, axis=-1, keepdims=True)
    var2 = jnp.mean((h2 - mu2) ** 2, axis=-1, keepdims=True)
    h2n = (h2 - mu2) / jnp.sqrt(var2 + 1e-5) * g2_ref[...] + be2_ref[...]
    h2r = jnp.maximum(h2n, 0.0).reshape(_MB_M, _K, _O)
    out_ref[0] = jnp.max(h2r, axis=1)


def _mlp(g, cent, w1x, g1, be1, w2, b2, g2, be2):
    B = g.shape[0]
    return pl.pallas_call(
        _mlp_body,
        grid=(B, _M // _MB_M),
        in_specs=[
            pl.BlockSpec((1, _MB_M, _K, _H), lambda b, m: (b, m, 0, 0)),
            pl.BlockSpec((1, _MB_M, 3), lambda b, m: (b, m, 0)),
            pl.BlockSpec((3, _H), lambda b, m: (0, 0)),
            pl.BlockSpec((1, _H), lambda b, m: (0, 0)),
            pl.BlockSpec((1, _H), lambda b, m: (0, 0)),
            pl.BlockSpec((_H, _O), lambda b, m: (0, 0)),
            pl.BlockSpec((1, _O), lambda b, m: (0, 0)),
            pl.BlockSpec((1, _O), lambda b, m: (0, 0)),
            pl.BlockSpec((1, _O), lambda b, m: (0, 0)),
        ],
        out_specs=pl.BlockSpec((1, _MB_M, _O), lambda b, m: (b, m, 0)),
        out_shape=jax.ShapeDtypeStruct((B, _M, _O), jnp.float32),
    )(g, cent, w1x, g1, be1, w2, b2, g2, be2)


def kernel(xyz, feats, W1, b1, g1, be1, W2, b2, g2, be2):
    B = xyz.shape[0]
    xf = xyz.transpose(0, 2, 1)
    xt = xf.reshape(B, 3, 64, 128)
    cent, cidx = _fps(xyz, xt)
    nidx = _ballq(cent, cidx, xf)
    w1f = W1[:_H]
    w1x = W1[_H:]
    p = _pmat(feats, xyz, w1f, w1x, b1.reshape(1, _H))
    g = _sc_gather(p.reshape(B * _N, _H), nidx.reshape(-1))
    g = g.reshape(B, _M, _K, _H)
    pooled = _mlp(g, cent, w1x, g1.reshape(1, _H), be1.reshape(1, _H),
                  W2, b2.reshape(1, _O), g2.reshape(1, _O),
                  be2.reshape(1, _O))
    return (cent, pooled)


# speedup vs baseline: 5.0147x; 1.0610x over previous
import functools

import jax
import jax.numpy as jnp
from jax import lax
from jax.experimental import pallas as pl
from jax.experimental.pallas import tpu as pltpu
from jax.experimental.pallas import tpu_sc as plsc

_N = 8192
_M = 2048
_K = 32
_H = 128
_O = 256
_R2 = 0.1 * 0.1
_MB_Q = 8
_MB_M = 64
_NB = 512


def _fps_body(xyz_ref, xt_ref, cent_ref, cidx_ref, dist_ref):
    B = xyz_ref.shape[0]
    dist_ref[...] = jnp.full((B * 64, 128), jnp.inf, jnp.float32)
    rows = lax.broadcasted_iota(jnp.int32, (64, 128), 0)
    cols = lax.broadcasted_iota(jnp.int32, (64, 128), 1)
    idx2d = rows * 128 + cols

    def body(i, fars):
        new_fars = []
        for b in range(B):
            far = fars[b]
            cent = xyz_ref[b, pl.ds(far, 1), :]
            cent_ref[b, pl.ds(i, 1), :] = cent
            cidx_ref[b, pl.ds(i, 1), :] = jnp.full((1, 1), far, jnp.int32)
            c0 = cent[0, 0]
            c1 = cent[0, 1]
            c2 = cent[0, 2]
            dx = xt_ref[b, 0] - c0
            dy = xt_ref[b, 1] - c1
            dz = xt_ref[b, 2] - c2
            d = (dx * dx + dy * dy) + dz * dz
            nd = jnp.minimum(dist_ref[pl.ds(b * 64, 64), :], d)
            dist_ref[pl.ds(b * 64, 64), :] = nd
            m = jnp.max(nd)
            cand = jnp.where(nd == m, idx2d, _N)
            new_fars.append(jnp.min(cand))
        return tuple(new_fars)

    lax.fori_loop(0, _M, body, (jnp.int32(0),) * B)


def _fps(xyz, xt):
    B = xyz.shape[0]
    return pl.pallas_call(
        _fps_body,
        grid=(1,),
        in_specs=[
            pl.BlockSpec((B, _N, 3), lambda g: (0, 0, 0)),
            pl.BlockSpec((B, 3, 64, 128), lambda g: (0, 0, 0, 0)),
        ],
        out_specs=[
            pl.BlockSpec((B, _M, 3), lambda g: (0, 0, 0)),
            pl.BlockSpec((B, _M, 1), lambda g: (0, 0, 0)),
        ],
        out_shape=[
            jax.ShapeDtypeStruct((B, _M, 3), jnp.float32),
            jax.ShapeDtypeStruct((B, _M, 1), jnp.int32),
        ],
        scratch_shapes=[pltpu.VMEM((B * 64, 128), jnp.float32)],
    )(xyz, xt)


def _ballq_body(cent_ref, cidx_ref, xf_ref, nidx_ref):
    b = pl.program_id(0)
    c = cent_ref[0]
    c0 = c[:, 0:1]
    c1 = c[:, 1:2]
    c2 = c[:, 2:3]
    x0 = xf_ref[0, 0:1, :]
    x1 = xf_ref[0, 1:2, :]
    x2 = xf_ref[0, 2:3, :]
    cs = (c0 * c0 + c1 * c1) + c2 * c2
    xs = (x0 * x0 + x1 * x1) + x2 * x2
    dot = jnp.dot(c, xf_ref[0], preferred_element_type=jnp.float32)
    d2 = cs - 2.0 * dot + xs
    valid = d2 < _R2
    iota_n = lax.broadcasted_iota(jnp.int32, (_MB_Q, _N), 1)
    keys = jnp.where(valid, iota_n, _N)
    cols = []
    for _ in range(_K):
        cur = jnp.min(keys, axis=1, keepdims=True)
        cols.append(cur)
        keys = jnp.where(keys == cur, _N, keys)
    nidx = jnp.concatenate(cols, axis=1)
    nidx = jnp.where(nidx == _N, cidx_ref[0], nidx)
    nidx_ref[0] = nidx + b * _N


def _ballq(cent, cidx, xf):
    B = cent.shape[0]
    return pl.pallas_call(
        _ballq_body,
        grid=(B, _M // _MB_Q),
        in_specs=[
            pl.BlockSpec((1, _MB_Q, 3), lambda b, m: (b, m, 0)),
            pl.BlockSpec((1, _MB_Q, 1), lambda b, m: (b, m, 0)),
            pl.BlockSpec((1, 3, _N), lambda b, m: (b, 0, 0)),
        ],
        out_specs=pl.BlockSpec((1, _MB_Q, _K), lambda b, m: (b, m, 0)),
        out_shape=jax.ShapeDtypeStruct((B, _M, _K), jnp.int32),
    )(cent, cidx, xf)


def _pmat_body(f_ref, x_ref, w1f_ref, w1x_ref, b1_ref, p_ref):
    xb = x_ref[0]
    px = (xb[:, 0:1] * w1x_ref[0:1, :]
          + xb[:, 1:2] * w1x_ref[1:2, :]
          + xb[:, 2:3] * w1x_ref[2:3, :])
    p_ref[0] = (jnp.dot(f_ref[0], w1f_ref[...],
                        preferred_element_type=jnp.float32)
                + px + b1_ref[...])


def _pmat(feats, xyz, w1f, w1x, b1_2d):
    B = feats.shape[0]
    return pl.pallas_call(
        _pmat_body,
        grid=(B, _N // _NB),
        in_specs=[
            pl.BlockSpec((1, _NB, _H), lambda b, n: (b, n, 0)),
            pl.BlockSpec((1, _NB, 3), lambda b, n: (b, n, 0)),
            pl.BlockSpec((_H, _H), lambda b, n: (0, 0)),
            pl.BlockSpec((3, _H), lambda b, n: (0, 0)),
            pl.BlockSpec((1, _H), lambda b, n: (0, 0)),
        ],
        out_specs=pl.BlockSpec((1, _NB, _H), lambda b, n: (b, n, 0)),
        out_shape=jax.ShapeDtypeStruct((B, _N, _H), jnp.float32),
    )(feats, xyz, w1f, w1x, b1_2d)


def _sc_gather(table, idx):
    R = idx.shape[0]
    NW = 32
    per_w = R // NW
    ch = 128
    iters = per_w // ch
    mesh = plsc.VectorSubcoreMesh(core_axis_name="c", subcore_axis_name="s")

    @functools.partial(
        pl.kernel,
        mesh=mesh,
        out_type=jax.ShapeDtypeStruct((R, _H), jnp.float32),
        scratch_types=[
            pltpu.VMEM((ch,), jnp.int32),
            pltpu.VMEM((ch, _H), jnp.float32),
            pltpu.SemaphoreType.DMA,
        ],
    )
    def k(table_hbm, idx_hbm, out_hbm, idx_v, rows_v, sem):
        wid = lax.axis_index("s") * 2 + lax.axis_index("c")
        base = wid * per_w

        def body(i, carry):
            o = base + i * ch
            pltpu.sync_copy(idx_hbm.at[pl.ds(o, ch)], idx_v)
            pltpu.async_copy(table_hbm.at[idx_v], rows_v, sem).wait()
            pltpu.sync_copy(rows_v, out_hbm.at[pl.ds(o, ch)])
            return carry

        lax.fori_loop(0, iters, body, jnp.int32(0))

    return k(table, idx)


def _mlp_body(g_ref, c_ref, w1x_ref, g1_ref, be1_ref, w2_ref, b2_ref,
              g2_ref, be2_ref, out_ref):
    c = c_ref[0]
    q = (c[:, 0:1] * w1x_ref[0:1, :]
         + c[:, 1:2] * w1x_ref[1:2, :]
         + c[:, 2:3] * w1x_ref[2:3, :])
    x = g_ref[0] - q[:, None, :]
    mu = jnp.mean(x, axis=-1, keepdims=True)
    var = jnp.mean((x - mu) ** 2, axis=-1, keepdims=True)
    xn = (x - mu) / jnp.sqrt(var + 1e-5) * g1_ref[...] + be1_ref[...]
    h = jnp.maximum(xn, 0.0)
    h2 = (jnp.dot(h.reshape(_MB_M * _K, _H), w2_ref[...],
                  preferred_element_type=jnp.float32) + b2_ref[...])
    mu2 = jnp.mean(h2, axis=-1, keepdims=True)
    var2 = jnp.mean((h2 - mu2) ** 2, axis=-1, keepdims=True)
    h2n = (h2 - mu2) / jnp.sqrt(var2 + 1e-5) * g2_ref[...] + be2_ref[...]
    h2r = jnp.maximum(h2n, 0.0).reshape(_MB_M, _K, _O)
    out_ref[0] = jnp.max(h2r, axis=1)


def _mlp(g, cent, w1x, g1, be1, w2, b2, g2, be2):
    B = g.shape[0]
    return pl.pallas_call(
        _mlp_body,
        grid=(B, _M // _MB_M),
        in_specs=[
            pl.BlockSpec((1, _MB_M, _K, _H), lambda b, m: (b, m, 0, 0)),
            pl.BlockSpec((1, _MB_M, 3), lambda b, m: (b, m, 0)),
            pl.BlockSpec((3, _H), lambda b, m: (0, 0)),
            pl.BlockSpec((1, _H), lambda b, m: (0, 0)),
            pl.BlockSpec((1, _H), lambda b, m: (0, 0)),
            pl.BlockSpec((_H, _O), lambda b, m: (0, 0)),
            pl.BlockSpec((1, _O), lambda b, m: (0, 0)),
            pl.BlockSpec((1, _O), lambda b, m: (0, 0)),
            pl.BlockSpec((1, _O), lambda b, m: (0, 0)),
        ],
        out_specs=pl.BlockSpec((1, _MB_M, _O), lambda b, m: (b, m, 0)),
        out_shape=jax.ShapeDtypeStruct((B, _M, _O), jnp.float32),
    )(g, cent, w1x, g1, be1, w2, b2, g2, be2)


def kernel(xyz, feats, W1, b1, g1, be1, W2, b2, g2, be2):
    B = xyz.shape[0]
    xf = xyz.transpose(0, 2, 1)
    xt = xf.reshape(B, 3, 64, 128)
    cent, cidx = _fps(xyz, xt)
    nidx = _ballq(cent, cidx, xf)
    w1f = W1[:_H]
    w1x = W1[_H:]
    p = _pmat(feats, xyz, w1f, w1x, b1.reshape(1, _H))
    g = _sc_gather(p.reshape(B * _N, _H), nidx.reshape(-1))
    g = g.reshape(B, _M, _K, _H)
    pooled = _mlp(g, cent, w1x, g1.reshape(1, _H), be1.reshape(1, _H),
                  W2, b2.reshape(1, _O), g2.reshape(1, _O),
                  be2.reshape(1, _O))
    return (cent, pooled)
